# baseline (speedup 1.0000x reference)
import functools

import jax
import jax.numpy as jnp
from jax import lax
from jax.experimental import pallas as pl
from jax.experimental.pallas import tpu as pltpu
from jax.experimental.pallas import tpu_sc as plsc

N = 10000
E = 320000
D_IN = 128
D_H = 64
N_CLASSES = 2
N_GRAPHS = 16
EPS = 1e-5

NC = 2
NS = 16
CHUNK = 128
K0 = 136
K1 = 24
E_PAD = NS * (K0 + K1) * CHUNK
E0 = NS * K0 * CHUNK
ROWS_PER_TILE = 640
N_PAD = NS * ROWS_PER_TILE

@functools.cache
def _sc_kernels():
    mesh = plsc.VectorSubcoreMesh(core_axis_name="c", subcore_axis_name="s",
                                  num_cores=NC, num_subcores=NS)

    @functools.partial(
        pl.kernel,
        out_type=jax.ShapeDtypeStruct((NC, N_PAD, 16), jnp.float32),
        mesh=mesh,
        scratch_types=[
            pltpu.VMEM_SHARED((N_PAD, 16), jnp.float32),
            pltpu.VMEM((K0, CHUNK), jnp.int32),
            pltpu.VMEM((CHUNK, 16), jnp.float32),
            pltpu.SemaphoreType.DMA,
        ],
        compiler_params=pltpu.CompilerParams(use_tc_tiling_on_sc=False),
    )
    def sc_degree(dst0_hbm, dst1_hbm, zeros_hbm, ones_hbm, out_hbm,
                  acc, dstb_v, ones_v, semd):
        c = lax.axis_index("c")
        s = lax.axis_index("s")
        row0 = pl.multiple_of(s * ROWS_PER_TILE, 8)
        pltpu.sync_copy(zeros_hbm.at[pl.ds(row0, ROWS_PER_TILE)],
                        acc.at[pl.ds(row0, ROWS_PER_TILE)])
        pltpu.sync_copy(ones_hbm, ones_v)

        def run(dst_hbm, k):
            pltpu.sync_copy(dst_hbm.at[s], dstb_v.at[pl.ds(0, k)])
            plsc.subcore_barrier()

            def group(gi, carry):
                for b in range(8):
                    pltpu.async_copy(ones_v, acc.at[dstb_v.at[gi * 8 + b]],
                                     semd, add=True)
                for b in range(8):
                    pltpu.make_async_copy(zeros_hbm.at[pl.ds(0, CHUNK)],
                                          ones_v, semd).wait()
                return carry

            lax.fori_loop(0, k // 8, group, 0)

        @pl.when(c == 0)
        def _():
            run(dst0_hbm, K0)

        @pl.when(c == 1)
        def _():
            run(dst1_hbm, K1)

        plsc.subcore_barrier()
        pltpu.sync_copy(acc.at[pl.ds(row0, ROWS_PER_TILE)],
                        out_hbm.at[c, pl.ds(row0, ROWS_PER_TILE)])

    @functools.partial(
        pl.kernel,
        out_type=jax.ShapeDtypeStruct((NC, N_PAD, D_H), jnp.float32),
        mesh=mesh,
        scratch_types=[
            pltpu.VMEM_SHARED((N_PAD, D_H), jnp.float32),
            pltpu.VMEM((K0, CHUNK), jnp.int32),
            pltpu.VMEM((K0, CHUNK), jnp.int32),
            pltpu.VMEM((CHUNK, D_H), jnp.float32),
            pltpu.VMEM((CHUNK, D_H), jnp.float32),
            pltpu.SemaphoreType.DMA,
            pltpu.SemaphoreType.DMA,
        ],
        compiler_params=pltpu.CompilerParams(use_tc_tiling_on_sc=False),
    )
    def sc_scatter(g_hbm, src0_hbm, dst0_hbm, src1_hbm, dst1_hbm,
                   zeros_hbm, out_hbm,
                   acc, srcb_v, dstb_v, rows0, rows1, sem0, sem1):
        c = lax.axis_index("c")
        s = lax.axis_index("s")
        row0 = pl.multiple_of(s * ROWS_PER_TILE, 8)

        def zrow(i, carry):
            for q in range(D_H // 16):
                rows0[i, pl.ds(q * 16, 16)] = jnp.zeros((16,), jnp.float32)
            return carry

        lax.fori_loop(0, CHUNK, zrow, 0)
        for q in range(ROWS_PER_TILE // CHUNK):
            pltpu.sync_copy(rows0, acc.at[pl.ds(row0 + q * CHUNK, CHUNK)])

        def run(src_hbm, dst_hbm, k):
            pltpu.sync_copy(src_hbm.at[s], srcb_v.at[pl.ds(0, k)])
            pltpu.sync_copy(dst_hbm.at[s], dstb_v.at[pl.ds(0, k)])
            plsc.subcore_barrier()

            pltpu.async_copy(g_hbm.at[srcb_v.at[0]], rows0, sem0)
            pltpu.async_copy(g_hbm.at[srcb_v.at[1]], rows1, sem1)

            def pair(jj, carry):
                for b, (rows, sem) in enumerate(((rows0, sem0),
                                                 (rows1, sem1))):
                    j = jj * 2 + b
                    pltpu.make_async_copy(zeros_hbm.at[pl.ds(0, CHUNK)],
                                          rows, sem).wait()
                    pltpu.sync_copy(rows, acc.at[dstb_v.at[j]], add=True)

                    @pl.when(j + 2 < k)
                    def _():
                        pltpu.async_copy(g_hbm.at[srcb_v.at[j + 2]], rows,
                                         sem)
                return carry

            lax.fori_loop(0, k // 2, pair, 0)

        @pl.when(c == 0)
        def _():
            run(src0_hbm, dst0_hbm, K0)

        @pl.when(c == 1)
        def _():
            run(src1_hbm, dst1_hbm, K1)

        plsc.subcore_barrier()
        pltpu.sync_copy(acc.at[pl.ds(row0, ROWS_PER_TILE)],
                        out_hbm.at[c, pl.ds(row0, ROWS_PER_TILE)])

    return sc_degree, sc_scatter


def _dot_t(a, w):
    return lax.dot_general(a, w, (((1,), (1,)), ((), ())),
                           precision=lax.Precision.HIGHEST,
                           preferred_element_type=jnp.float32)


def _bn_rows(h, gamma, beta):
    mu = jnp.mean(h, axis=0)
    var = jnp.mean((h - mu) ** 2, axis=0)
    return (h - mu) * lax.rsqrt(var + EPS) * gamma + beta


def _tc_prep_body(x_ref, bng_ref, bnb_ref, w0_ref, degp_ref,
                  g_ref, dinv_ref):
    x = x_ref[...]
    h = _bn_rows(x, bng_ref[...], bnb_ref[...])
    h1 = _dot_t(h, w0_ref[...])
    deg = degp_ref[0, :, 0:1] + degp_ref[1, :, 0:1] + 1.0
    dinv = lax.rsqrt(deg)
    g_ref[...] = jnp.concatenate(
        [h1 * dinv[:N], jnp.zeros((N_PAD - N, D_H), jnp.float32)], axis=0)
    dinv_ref[...] = dinv


_tc_prep = pl.pallas_call(
    _tc_prep_body,
    out_shape=(jax.ShapeDtypeStruct((N_PAD, D_H), jnp.float32),
               jax.ShapeDtypeStruct((N_PAD, 1), jnp.float32)),
)


def _tc_mid_body(p_ref, gprev_ref, dinv_ref, b_ref, gam_ref, bet_ref, w_ref,
                 g_ref):
    dinv = dinv_ref[...]
    acc = p_ref[0, :N, :] + p_ref[1, :N, :] + gprev_ref[:N, :]
    out = acc * dinv[:N] + b_ref[...]
    h = jnp.maximum(_bn_rows(out, gam_ref[...], bet_ref[...]), 0.0)
    h1 = _dot_t(h, w_ref[...])
    g_ref[...] = jnp.concatenate(
        [h1 * dinv[:N], jnp.zeros((N_PAD - N, D_H), jnp.float32)], axis=0)


_tc_mid = pl.pallas_call(
    _tc_mid_body,
    out_shape=jax.ShapeDtypeStruct((N_PAD, D_H), jnp.float32),
)


def _tc_final_body(p_ref, gprev_ref, dinv_ref, b_ref, gam_ref, bet_ref,
                   batch_ref, cw1_ref, cb1_ref, cw2_ref, cb2_ref, res_ref):
    dinv = dinv_ref[...]
    acc = p_ref[0, :N, :] + p_ref[1, :N, :] + gprev_ref[:N, :]
    out = acc * dinv[:N] + b_ref[...]
    h = jnp.maximum(_bn_rows(out, gam_ref[...], bet_ref[...]), 0.0)
    seg = batch_ref[...]
    oh = (seg == lax.broadcasted_iota(jnp.int32, (1, N_GRAPHS), 1))
    oh = oh.astype(jnp.float32)
    sums = lax.dot_general(oh, h, (((0,), (0,)), ((), ())),
                           precision=lax.Precision.HIGHEST,
                           preferred_element_type=jnp.float32)
    cnt = jnp.sum(oh, axis=0)[:, None]
    pooled = sums / jnp.maximum(cnt, 1.0)
    hc = jnp.maximum(_dot_t(pooled, cw1_ref[...]) + cb1_ref[...], 0.0)
    res_ref[...] = _dot_t(hc, cw2_ref[...]) + cb2_ref[...]


_tc_final = pl.pallas_call(
    _tc_final_body,
    out_shape=jax.ShapeDtypeStruct((N_GRAPHS, N_CLASSES), jnp.float32),
)


def kernel(x, edge_index, batch, bn_in_g, bn_in_b, W0, b0, g0, be0,
           W1, b1, g1, be1, W2, b2, g2, be2, cW1, cb1, cW2, cb2):
    pad = E_PAD - E
    src = jnp.concatenate([edge_index[0], jnp.full((pad,), N, jnp.int32)])
    dst = jnp.concatenate([edge_index[1], jnp.full((pad,), N, jnp.int32)])
    src0 = src[:E0].reshape(NS, K0, CHUNK)
    dst0 = dst[:E0].reshape(NS, K0, CHUNK)
    src1 = src[E0:].reshape(NS, K1, CHUNK)
    dst1 = dst[E0:].reshape(NS, K1, CHUNK)

    zeros16 = jnp.zeros((N_PAD, 16), jnp.float32)
    ones16 = jnp.ones((CHUNK, 16), jnp.float32)
    zeros64 = jnp.zeros((N_PAD, D_H), jnp.float32)
    batch2d = batch.reshape(N, 1)

    sc_degree, sc_scatter = _sc_kernels()
    degp = sc_degree(dst0, dst1, zeros16, ones16)
    gfeat, dinv = _tc_prep(x, bn_in_g, bn_in_b, W0, degp)

    p = sc_scatter(gfeat, src0, dst0, src1, dst1, zeros64)
    gfeat = _tc_mid(p, gfeat, dinv, b0, g0, be0, W1)

    p = sc_scatter(gfeat, src0, dst0, src1, dst1, zeros64)
    gfeat = _tc_mid(p, gfeat, dinv, b1, g1, be1, W2)

    p = sc_scatter(gfeat, src0, dst0, src1, dst1, zeros64)
    return _tc_final(p, gfeat, dinv, b2, g2, be2, batch2d, cW1, cb1, cW2, cb2)


# speedup vs baseline: 1.6844x; 1.6844x over previous
import functools

import jax
import jax.numpy as jnp
from jax import lax
from jax.experimental import pallas as pl
from jax.experimental.pallas import tpu as pltpu
from jax.experimental.pallas import tpu_sc as plsc

N = 10000
E = 320000
D_IN = 128
D_H = 64
N_CLASSES = 2
N_GRAPHS = 16
EPS = 1e-5

NC = 2
NS = 16
CHUNK = 128
K0 = 96
K1 = 64
E_PAD = NS * (K0 + K1) * CHUNK
E0 = NS * K0 * CHUNK
ROWS_PER_TILE = 640
N_PAD = NS * ROWS_PER_TILE

@functools.cache
def _sc_kernels():
    mesh = plsc.VectorSubcoreMesh(core_axis_name="c", subcore_axis_name="s",
                                  num_cores=NC, num_subcores=NS)

    @functools.partial(
        pl.kernel,
        out_type=jax.ShapeDtypeStruct((NC, N_PAD, 16), jnp.float32),
        mesh=mesh,
        scratch_types=[
            pltpu.VMEM_SHARED((N_PAD, 16), jnp.float32),
            pltpu.VMEM((K0, CHUNK), jnp.int32),
            pltpu.VMEM((CHUNK, 16), jnp.float32),
            pltpu.SemaphoreType.DMA,
        ],
        compiler_params=pltpu.CompilerParams(use_tc_tiling_on_sc=False),
    )
    def sc_degree(dst0_hbm, dst1_hbm, zeros_hbm, ones_hbm, out_hbm,
                  acc, dstb_v, ones_v, semd):
        c = lax.axis_index("c")
        s = lax.axis_index("s")
        row0 = pl.multiple_of(s * ROWS_PER_TILE, 8)
        pltpu.sync_copy(zeros_hbm.at[pl.ds(row0, ROWS_PER_TILE)],
                        acc.at[pl.ds(row0, ROWS_PER_TILE)])
        pltpu.sync_copy(ones_hbm, ones_v)

        def run(dst_hbm, k):
            pltpu.sync_copy(dst_hbm.at[s], dstb_v.at[pl.ds(0, k)])
            plsc.subcore_barrier()

            def group(gi, carry):
                for b in range(8):
                    pltpu.async_copy(ones_v, acc.at[dstb_v.at[gi * 8 + b]],
                                     semd, add=True)
                for b in range(8):
                    pltpu.make_async_copy(zeros_hbm.at[pl.ds(0, CHUNK)],
                                          ones_v, semd).wait()
                return carry

            lax.fori_loop(0, k // 8, group, 0)

        @pl.when(c == 0)
        def _():
            run(dst0_hbm, K0)

        @pl.when(c == 1)
        def _():
            run(dst1_hbm, K1)

        plsc.subcore_barrier()
        pltpu.sync_copy(acc.at[pl.ds(row0, ROWS_PER_TILE)],
                        out_hbm.at[c, pl.ds(row0, ROWS_PER_TILE)])

    @functools.partial(
        pl.kernel,
        out_type=jax.ShapeDtypeStruct((NC, N_PAD, D_H), jnp.float32),
        mesh=mesh,
        scratch_types=[
            pltpu.VMEM_SHARED((N_PAD, D_H), jnp.float32),
            pltpu.VMEM_SHARED((N_PAD, D_H), jnp.float32),
            pltpu.VMEM((K0, CHUNK), jnp.int32),
            pltpu.VMEM((K0, CHUNK), jnp.int32),
            pltpu.VMEM((CHUNK, D_H), jnp.float32),
            pltpu.VMEM((CHUNK, D_H), jnp.float32),
            pltpu.SemaphoreType.DMA,
            pltpu.SemaphoreType.DMA,
        ],
        compiler_params=pltpu.CompilerParams(use_tc_tiling_on_sc=False),
    )
    def sc_scatter(g_hbm, src0_hbm, dst0_hbm, src1_hbm, dst1_hbm,
                   zeros_hbm, out_hbm,
                   acc, gbuf, srcb_v, dstb_v, rows0, rows1, sem0, sem1):
        c = lax.axis_index("c")
        s = lax.axis_index("s")
        row0 = pl.multiple_of(s * ROWS_PER_TILE, 8)

        def zrow(i, carry):
            for q in range(D_H // 16):
                rows0[i, pl.ds(q * 16, 16)] = jnp.zeros((16,), jnp.float32)
            return carry

        lax.fori_loop(0, CHUNK, zrow, 0)
        for q in range(ROWS_PER_TILE // CHUNK):
            pltpu.sync_copy(rows0, acc.at[pl.ds(row0 + q * CHUNK, CHUNK)])
        pltpu.sync_copy(g_hbm.at[pl.ds(row0, ROWS_PER_TILE)],
                        gbuf.at[pl.ds(row0, ROWS_PER_TILE)])

        def run(src_hbm, dst_hbm, k):
            pltpu.sync_copy(src_hbm.at[s], srcb_v.at[pl.ds(0, k)])
            pltpu.sync_copy(dst_hbm.at[s], dstb_v.at[pl.ds(0, k)])
            plsc.subcore_barrier()

            pltpu.async_copy(gbuf.at[srcb_v.at[0]], rows0, sem0)
            pltpu.async_copy(gbuf.at[srcb_v.at[1]], rows1, sem1)

            def pair(jj, carry):
                for b, (rows, sem) in enumerate(((rows0, sem0),
                                                 (rows1, sem1))):
                    j = jj * 2 + b
                    pltpu.make_async_copy(zeros_hbm.at[pl.ds(0, CHUNK)],
                                          rows, sem).wait()
                    pltpu.sync_copy(rows, acc.at[dstb_v.at[j]], add=True)

                    @pl.when(j + 2 < k)
                    def _():
                        pltpu.async_copy(gbuf.at[srcb_v.at[j + 2]], rows,
                                         sem)
                return carry

            lax.fori_loop(0, k // 2, pair, 0)

        @pl.when(c == 0)
        def _():
            run(src0_hbm, dst0_hbm, K0)

        @pl.when(c == 1)
        def _():
            run(src1_hbm, dst1_hbm, K1)

        plsc.subcore_barrier()
        pltpu.sync_copy(acc.at[pl.ds(row0, ROWS_PER_TILE)],
                        out_hbm.at[c, pl.ds(row0, ROWS_PER_TILE)])

    return sc_degree, sc_scatter


def _dot_t(a, w):
    return lax.dot_general(a, w, (((1,), (1,)), ((), ())),
                           precision=lax.Precision.HIGHEST,
                           preferred_element_type=jnp.float32)


def _bn_rows(h, gamma, beta):
    mu = jnp.mean(h, axis=0)
    var = jnp.mean((h - mu) ** 2, axis=0)
    return (h - mu) * lax.rsqrt(var + EPS) * gamma + beta


def _tc_prep_body(x_ref, bng_ref, bnb_ref, w0_ref, degp_ref,
                  g_ref, dinv_ref):
    x = x_ref[...]
    h = _bn_rows(x, bng_ref[...], bnb_ref[...])
    h1 = _dot_t(h, w0_ref[...])
    deg = degp_ref[0, :, 0:1] + degp_ref[1, :, 0:1] + 1.0
    dinv = lax.rsqrt(deg)
    g_ref[...] = jnp.concatenate(
        [h1 * dinv[:N], jnp.zeros((N_PAD - N, D_H), jnp.float32)], axis=0)
    dinv_ref[...] = dinv


_tc_prep = pl.pallas_call(
    _tc_prep_body,
    out_shape=(jax.ShapeDtypeStruct((N_PAD, D_H), jnp.float32),
               jax.ShapeDtypeStruct((N_PAD, 1), jnp.float32)),
)


def _tc_mid_body(p_ref, gprev_ref, dinv_ref, b_ref, gam_ref, bet_ref, w_ref,
                 g_ref):
    dinv = dinv_ref[...]
    acc = p_ref[0, :N, :] + p_ref[1, :N, :] + gprev_ref[:N, :]
    out = acc * dinv[:N] + b_ref[...]
    h = jnp.maximum(_bn_rows(out, gam_ref[...], bet_ref[...]), 0.0)
    h1 = _dot_t(h, w_ref[...])
    g_ref[...] = jnp.concatenate(
        [h1 * dinv[:N], jnp.zeros((N_PAD - N, D_H), jnp.float32)], axis=0)


_tc_mid = pl.pallas_call(
    _tc_mid_body,
    out_shape=jax.ShapeDtypeStruct((N_PAD, D_H), jnp.float32),
)


def _tc_final_body(p_ref, gprev_ref, dinv_ref, b_ref, gam_ref, bet_ref,
                   batch_ref, cw1_ref, cb1_ref, cw2_ref, cb2_ref, res_ref):
    dinv = dinv_ref[...]
    acc = p_ref[0, :N, :] + p_ref[1, :N, :] + gprev_ref[:N, :]
    out = acc * dinv[:N] + b_ref[...]
    h = jnp.maximum(_bn_rows(out, gam_ref[...], bet_ref[...]), 0.0)
    seg = batch_ref[...]
    oh = (seg == lax.broadcasted_iota(jnp.int32, (1, N_GRAPHS), 1))
    oh = oh.astype(jnp.float32)
    sums = lax.dot_general(oh, h, (((0,), (0,)), ((), ())),
                           precision=lax.Precision.HIGHEST,
                           preferred_element_type=jnp.float32)
    cnt = jnp.sum(oh, axis=0)[:, None]
    pooled = sums / jnp.maximum(cnt, 1.0)
    hc = jnp.maximum(_dot_t(pooled, cw1_ref[...]) + cb1_ref[...], 0.0)
    res_ref[...] = _dot_t(hc, cw2_ref[...]) + cb2_ref[...]


_tc_final = pl.pallas_call(
    _tc_final_body,
    out_shape=jax.ShapeDtypeStruct((N_GRAPHS, N_CLASSES), jnp.float32),
)


def kernel(x, edge_index, batch, bn_in_g, bn_in_b, W0, b0, g0, be0,
           W1, b1, g1, be1, W2, b2, g2, be2, cW1, cb1, cW2, cb2):
    pad = E_PAD - E
    src = jnp.concatenate([edge_index[0], jnp.full((pad,), N, jnp.int32)])
    dst = jnp.concatenate([edge_index[1], jnp.full((pad,), N, jnp.int32)])
    src0 = src[:E0].reshape(NS, K0, CHUNK)
    dst0 = dst[:E0].reshape(NS, K0, CHUNK)
    src1 = src[E0:].reshape(NS, K1, CHUNK)
    dst1 = dst[E0:].reshape(NS, K1, CHUNK)

    zeros16 = jnp.zeros((N_PAD, 16), jnp.float32)
    ones16 = jnp.ones((CHUNK, 16), jnp.float32)
    zeros64 = jnp.zeros((N_PAD, D_H), jnp.float32)
    batch2d = batch.reshape(N, 1)

    sc_degree, sc_scatter = _sc_kernels()
    degp = sc_degree(dst0, dst1, zeros16, ones16)
    gfeat, dinv = _tc_prep(x, bn_in_g, bn_in_b, W0, degp)

    p = sc_scatter(gfeat, src0, dst0, src1, dst1, zeros64)
    gfeat = _tc_mid(p, gfeat, dinv, b0, g0, be0, W1)

    p = sc_scatter(gfeat, src0, dst0, src1, dst1, zeros64)
    gfeat = _tc_mid(p, gfeat, dinv, b1, g1, be1, W2)

    p = sc_scatter(gfeat, src0, dst0, src1, dst1, zeros64)
    return _tc_final(p, gfeat, dinv, b2, g2, be2, batch2d, cW1, cb1, cW2, cb2)


# speedup vs baseline: 1.7410x; 1.0336x over previous
import functools

import jax
import jax.numpy as jnp
from jax import lax
from jax.experimental import pallas as pl
from jax.experimental.pallas import tpu as pltpu
from jax.experimental.pallas import tpu_sc as plsc

N = 10000
E = 320000
D_IN = 128
D_H = 64
N_CLASSES = 2
N_GRAPHS = 16
EPS = 1e-5

NC = 2
NS = 16
CHUNK = 128
K0 = 96
K1 = 64
E_PAD = NS * (K0 + K1) * CHUNK
E0 = NS * K0 * CHUNK
ROWS_PER_TILE = 640
N_PAD = NS * ROWS_PER_TILE

@functools.cache
def _sc_kernels():
    mesh = plsc.VectorSubcoreMesh(core_axis_name="c", subcore_axis_name="s",
                                  num_cores=NC, num_subcores=NS)

    @functools.partial(
        pl.kernel,
        out_type=jax.ShapeDtypeStruct((NC, N_PAD, 16), jnp.float32),
        mesh=mesh,
        scratch_types=[
            pltpu.VMEM_SHARED((N_PAD, 16), jnp.float32),
            pltpu.VMEM((K0, CHUNK), jnp.int32),
            pltpu.VMEM((CHUNK, 16), jnp.float32),
            pltpu.SemaphoreType.DMA,
        ],
        compiler_params=pltpu.CompilerParams(use_tc_tiling_on_sc=False),
    )
    def sc_degree(dst0_hbm, dst1_hbm, zeros_hbm, ones_hbm, out_hbm,
                  acc, dstb_v, ones_v, semd):
        c = lax.axis_index("c")
        s = lax.axis_index("s")
        row0 = pl.multiple_of(s * ROWS_PER_TILE, 8)
        pltpu.sync_copy(zeros_hbm.at[pl.ds(row0, ROWS_PER_TILE)],
                        acc.at[pl.ds(row0, ROWS_PER_TILE)])
        pltpu.sync_copy(ones_hbm, ones_v)

        def run(dst_hbm, k):
            pltpu.sync_copy(dst_hbm.at[s], dstb_v.at[pl.ds(0, k)])
            plsc.subcore_barrier()

            def group(gi, carry):
                for b in range(8):
                    pltpu.async_copy(ones_v, acc.at[dstb_v.at[gi * 8 + b]],
                                     semd, add=True)
                for b in range(8):
                    pltpu.make_async_copy(zeros_hbm.at[pl.ds(0, CHUNK)],
                                          ones_v, semd).wait()
                return carry

            lax.fori_loop(0, k // 8, group, 0)

        @pl.when(c == 0)
        def _():
            run(dst0_hbm, K0)

        @pl.when(c == 1)
        def _():
            run(dst1_hbm, K1)

        plsc.subcore_barrier()
        pltpu.sync_copy(acc.at[pl.ds(row0, ROWS_PER_TILE)],
                        out_hbm.at[c, pl.ds(row0, ROWS_PER_TILE)])

    @functools.partial(
        pl.kernel,
        out_type=jax.ShapeDtypeStruct((NC, N_PAD, D_H), jnp.float32),
        mesh=mesh,
        scratch_types=[
            pltpu.VMEM_SHARED((N_PAD, D_H), jnp.float32),
            pltpu.VMEM_SHARED((N_PAD, D_H), jnp.float32),
            pltpu.VMEM((K0, CHUNK), jnp.int32),
            pltpu.VMEM((K0, CHUNK), jnp.int32),
            pltpu.VMEM((CHUNK, D_H), jnp.float32),
            pltpu.VMEM((CHUNK, D_H), jnp.float32),
            pltpu.SemaphoreType.DMA,
            pltpu.SemaphoreType.DMA,
        ],
        compiler_params=pltpu.CompilerParams(use_tc_tiling_on_sc=False),
    )
    def sc_scatter(g_hbm, src0_hbm, dst0_hbm, src1_hbm, dst1_hbm,
                   zeros_hbm, out_hbm,
                   acc, gbuf, srcb_v, dstb_v, rows0, rows1, sem0, sem1):
        c = lax.axis_index("c")
        s = lax.axis_index("s")
        row0 = pl.multiple_of(s * ROWS_PER_TILE, 8)

        def zrow(i, carry):
            for q in range(D_H // 16):
                rows0[i, pl.ds(q * 16, 16)] = jnp.zeros((16,), jnp.float32)
            return carry

        lax.fori_loop(0, CHUNK, zrow, 0)
        for q in range(ROWS_PER_TILE // CHUNK):
            pltpu.sync_copy(rows0, acc.at[pl.ds(row0 + q * CHUNK, CHUNK)])
        pltpu.sync_copy(g_hbm.at[pl.ds(row0, ROWS_PER_TILE)],
                        gbuf.at[pl.ds(row0, ROWS_PER_TILE)])

        def run(src_hbm, dst_hbm, k):
            pltpu.sync_copy(src_hbm.at[s], srcb_v.at[pl.ds(0, k)])
            pltpu.sync_copy(dst_hbm.at[s], dstb_v.at[pl.ds(0, k)])
            plsc.subcore_barrier()

            pltpu.async_copy(gbuf.at[srcb_v.at[0]], rows0, sem0)
            pltpu.async_copy(gbuf.at[srcb_v.at[1]], rows1, sem1)

            def pair(jj, carry):
                for b, (rows, sem) in enumerate(((rows0, sem0),
                                                 (rows1, sem1))):
                    j = jj * 2 + b
                    pltpu.make_async_copy(zeros_hbm.at[pl.ds(0, CHUNK)],
                                          rows, sem).wait()
                    pltpu.sync_copy(rows, acc.at[dstb_v.at[j]], add=True)

                    @pl.when(j + 2 < k)
                    def _():
                        pltpu.async_copy(gbuf.at[srcb_v.at[j + 2]], rows,
                                         sem)
                return carry

            lax.fori_loop(0, k // 2, pair, 0)

        @pl.when(c == 0)
        def _():
            run(src0_hbm, dst0_hbm, K0)

        @pl.when(c == 1)
        def _():
            run(src1_hbm, dst1_hbm, K1)

        plsc.subcore_barrier()
        pltpu.sync_copy(acc.at[pl.ds(row0, ROWS_PER_TILE)],
                        out_hbm.at[c, pl.ds(row0, ROWS_PER_TILE)])

    return sc_degree, sc_scatter


def _dot_t(a, w):
    return lax.dot_general(a, w, (((1,), (1,)), ((), ())),
                           preferred_element_type=jnp.float32)


def _bn_rows(h, gamma, beta):
    mu = jnp.mean(h, axis=0)
    var = jnp.mean((h - mu) ** 2, axis=0)
    return (h - mu) * lax.rsqrt(var + EPS) * gamma + beta


def _tc_prep_body(x_ref, bng_ref, bnb_ref, w0_ref, degp_ref,
                  g_ref, dinv_ref):
    x = x_ref[...]
    h = _bn_rows(x, bng_ref[...], bnb_ref[...])
    h1 = _dot_t(h, w0_ref[...])
    deg = degp_ref[0, :, 0:1] + degp_ref[1, :, 0:1] + 1.0
    dinv = lax.rsqrt(deg)
    g_ref[...] = jnp.concatenate(
        [h1 * dinv[:N], jnp.zeros((N_PAD - N, D_H), jnp.float32)], axis=0)
    dinv_ref[...] = dinv


_tc_prep = pl.pallas_call(
    _tc_prep_body,
    out_shape=(jax.ShapeDtypeStruct((N_PAD, D_H), jnp.float32),
               jax.ShapeDtypeStruct((N_PAD, 1), jnp.float32)),
)


def _tc_mid_body(p_ref, gprev_ref, dinv_ref, b_ref, gam_ref, bet_ref, w_ref,
                 g_ref):
    dinv = dinv_ref[...]
    acc = p_ref[0, :N, :] + p_ref[1, :N, :] + gprev_ref[:N, :]
    out = acc * dinv[:N] + b_ref[...]
    h = jnp.maximum(_bn_rows(out, gam_ref[...], bet_ref[...]), 0.0)
    h1 = _dot_t(h, w_ref[...])
    g_ref[...] = jnp.concatenate(
        [h1 * dinv[:N], jnp.zeros((N_PAD - N, D_H), jnp.float32)], axis=0)


_tc_mid = pl.pallas_call(
    _tc_mid_body,
    out_shape=jax.ShapeDtypeStruct((N_PAD, D_H), jnp.float32),
)


def _tc_final_body(p_ref, gprev_ref, dinv_ref, b_ref, gam_ref, bet_ref,
                   batch_ref, cw1_ref, cb1_ref, cw2_ref, cb2_ref, res_ref):
    dinv = dinv_ref[...]
    acc = p_ref[0, :N, :] + p_ref[1, :N, :] + gprev_ref[:N, :]
    out = acc * dinv[:N] + b_ref[...]
    h = jnp.maximum(_bn_rows(out, gam_ref[...], bet_ref[...]), 0.0)
    seg = batch_ref[...]
    oh = (seg == lax.broadcasted_iota(jnp.int32, (1, N_GRAPHS), 1))
    oh = oh.astype(jnp.float32)
    sums = lax.dot_general(oh, h, (((0,), (0,)), ((), ())),
                           precision=lax.Precision.HIGHEST,
                           preferred_element_type=jnp.float32)
    cnt = jnp.sum(oh, axis=0)[:, None]
    pooled = sums / jnp.maximum(cnt, 1.0)
    hc = jnp.maximum(_dot_t(pooled, cw1_ref[...]) + cb1_ref[...], 0.0)
    res_ref[...] = _dot_t(hc, cw2_ref[...]) + cb2_ref[...]


_tc_final = pl.pallas_call(
    _tc_final_body,
    out_shape=jax.ShapeDtypeStruct((N_GRAPHS, N_CLASSES), jnp.float32),
)


def kernel(x, edge_index, batch, bn_in_g, bn_in_b, W0, b0, g0, be0,
           W1, b1, g1, be1, W2, b2, g2, be2, cW1, cb1, cW2, cb2):
    pad = E_PAD - E
    src = jnp.concatenate([edge_index[0], jnp.full((pad,), N, jnp.int32)])
    dst = jnp.concatenate([edge_index[1], jnp.full((pad,), N, jnp.int32)])
    src0 = src[:E0].reshape(NS, K0, CHUNK)
    dst0 = dst[:E0].reshape(NS, K0, CHUNK)
    src1 = src[E0:].reshape(NS, K1, CHUNK)
    dst1 = dst[E0:].reshape(NS, K1, CHUNK)

    zeros16 = jnp.zeros((N_PAD, 16), jnp.float32)
    ones16 = jnp.ones((CHUNK, 16), jnp.float32)
    zeros64 = jnp.zeros((N_PAD, D_H), jnp.float32)
    batch2d = batch.reshape(N, 1)

    sc_degree, sc_scatter = _sc_kernels()
    degp = sc_degree(dst0, dst1, zeros16, ones16)
    gfeat, dinv = _tc_prep(x, bn_in_g, bn_in_b, W0, degp)

    p = sc_scatter(gfeat, src0, dst0, src1, dst1, zeros64)
    gfeat = _tc_mid(p, gfeat, dinv, b0, g0, be0, W1)

    p = sc_scatter(gfeat, src0, dst0, src1, dst1, zeros64)
    gfeat = _tc_mid(p, gfeat, dinv, b1, g1, be1, W2)

    p = sc_scatter(gfeat, src0, dst0, src1, dst1, zeros64)
    return _tc_final(p, gfeat, dinv, b2, g2, be2, batch2d, cW1, cb1, cW2, cb2)


# speedup vs baseline: 1.8839x; 1.0821x over previous
import functools

import jax
import jax.numpy as jnp
from jax import lax
from jax.experimental import pallas as pl
from jax.experimental.pallas import tpu as pltpu
from jax.experimental.pallas import tpu_sc as plsc

N = 10000
E = 320000
D_IN = 128
D_H = 64
N_CLASSES = 2
N_GRAPHS = 16
EPS = 1e-5

NC = 2
NS = 16
CHUNK = 128
K0 = 80
K1 = 80
E_PAD = NS * (K0 + K1) * CHUNK
E0 = NS * K0 * CHUNK
ROWS_PER_TILE = 640
N_PAD = NS * ROWS_PER_TILE

@functools.cache
def _sc_kernels():
    mesh = plsc.VectorSubcoreMesh(core_axis_name="c", subcore_axis_name="s",
                                  num_cores=NC, num_subcores=NS)

    @functools.partial(
        pl.kernel,
        out_type=jax.ShapeDtypeStruct((NC, N_PAD, 16), jnp.float32),
        mesh=mesh,
        scratch_types=[
            pltpu.VMEM_SHARED((N_PAD, 16), jnp.float32),
            pltpu.VMEM((K0, CHUNK), jnp.int32),
            pltpu.VMEM((CHUNK, 16), jnp.float32),
            pltpu.SemaphoreType.DMA,
        ],
        compiler_params=pltpu.CompilerParams(use_tc_tiling_on_sc=False),
    )
    def sc_degree(dst0_hbm, dst1_hbm, zeros_hbm, ones_hbm, out_hbm,
                  acc, dstb_v, ones_v, semd):
        c = lax.axis_index("c")
        s = lax.axis_index("s")
        row0 = pl.multiple_of(s * ROWS_PER_TILE, 8)
        pltpu.sync_copy(zeros_hbm.at[pl.ds(row0, ROWS_PER_TILE)],
                        acc.at[pl.ds(row0, ROWS_PER_TILE)])
        pltpu.sync_copy(ones_hbm, ones_v)

        def run(dst_hbm, k):
            pltpu.sync_copy(dst_hbm.at[s], dstb_v.at[pl.ds(0, k)])
            plsc.subcore_barrier()

            def group(gi, carry):
                for b in range(8):
                    pltpu.async_copy(ones_v, acc.at[dstb_v.at[gi * 8 + b]],
                                     semd, add=True)
                for b in range(8):
                    pltpu.make_async_copy(zeros_hbm.at[pl.ds(0, CHUNK)],
                                          ones_v, semd).wait()
                return carry

            lax.fori_loop(0, k // 8, group, 0)

        @pl.when(c == 0)
        def _():
            run(dst0_hbm, K0)

        @pl.when(c == 1)
        def _():
            run(dst1_hbm, K1)

        plsc.subcore_barrier()
        pltpu.sync_copy(acc.at[pl.ds(row0, ROWS_PER_TILE)],
                        out_hbm.at[c, pl.ds(row0, ROWS_PER_TILE)])

    @functools.partial(
        pl.kernel,
        out_type=jax.ShapeDtypeStruct((NC, N_PAD, D_H), jnp.float32),
        mesh=mesh,
        scratch_types=[
            pltpu.VMEM_SHARED((N_PAD, D_H), jnp.float32),
            pltpu.VMEM_SHARED((N_PAD, D_H), jnp.float32),
            pltpu.VMEM((K0, CHUNK), jnp.int32),
            pltpu.VMEM((K0, CHUNK), jnp.int32),
            pltpu.VMEM((CHUNK, D_H), jnp.float32),
            pltpu.VMEM((CHUNK, D_H), jnp.float32),
            pltpu.SemaphoreType.DMA,
            pltpu.SemaphoreType.DMA,
        ],
        compiler_params=pltpu.CompilerParams(use_tc_tiling_on_sc=False),
    )
    def sc_scatter(g_hbm, src0_hbm, dst0_hbm, src1_hbm, dst1_hbm,
                   zeros_hbm, out_hbm,
                   acc, gbuf, srcb_v, dstb_v, rows0, rows1, sem0, sem1):
        c = lax.axis_index("c")
        s = lax.axis_index("s")
        row0 = pl.multiple_of(s * ROWS_PER_TILE, 8)

        def zrow(i, carry):
            for q in range(D_H // 16):
                rows0[i, pl.ds(q * 16, 16)] = jnp.zeros((16,), jnp.float32)
            return carry

        lax.fori_loop(0, CHUNK, zrow, 0)
        for q in range(ROWS_PER_TILE // CHUNK):
            pltpu.sync_copy(rows0, acc.at[pl.ds(row0 + q * CHUNK, CHUNK)])
        pltpu.sync_copy(g_hbm.at[pl.ds(row0, ROWS_PER_TILE)],
                        gbuf.at[pl.ds(row0, ROWS_PER_TILE)])

        def run(src_hbm, dst_hbm, k):
            pltpu.sync_copy(src_hbm.at[s], srcb_v.at[pl.ds(0, k)])
            pltpu.sync_copy(dst_hbm.at[s], dstb_v.at[pl.ds(0, k)])
            plsc.subcore_barrier()

            pltpu.async_copy(gbuf.at[srcb_v.at[0]], rows0, sem0)
            pltpu.async_copy(gbuf.at[srcb_v.at[1]], rows1, sem1)

            def pair(jj, carry):
                for b, (rows, sem) in enumerate(((rows0, sem0),
                                                 (rows1, sem1))):
                    j = jj * 2 + b
                    pltpu.make_async_copy(zeros_hbm.at[pl.ds(0, CHUNK)],
                                          rows, sem).wait()
                    pltpu.sync_copy(rows, acc.at[dstb_v.at[j]], add=True)

                    @pl.when(j + 2 < k)
                    def _():
                        pltpu.async_copy(gbuf.at[srcb_v.at[j + 2]], rows,
                                         sem)
                return carry

            lax.fori_loop(0, k // 2, pair, 0)

        @pl.when(c == 0)
        def _():
            run(src0_hbm, dst0_hbm, K0)

        @pl.when(c == 1)
        def _():
            run(src1_hbm, dst1_hbm, K1)

        plsc.subcore_barrier()
        pltpu.sync_copy(acc.at[pl.ds(row0, ROWS_PER_TILE)],
                        out_hbm.at[c, pl.ds(row0, ROWS_PER_TILE)])

    return sc_degree, sc_scatter


def _dot_t(a, w):
    return lax.dot_general(a, w, (((1,), (1,)), ((), ())),
                           preferred_element_type=jnp.float32)


def _bn_rows(h, gamma, beta):
    mu = jnp.mean(h, axis=0)
    var = jnp.mean((h - mu) ** 2, axis=0)
    return (h - mu) * lax.rsqrt(var + EPS) * gamma + beta


def _tc_prep_body(x_ref, bng_ref, bnb_ref, w0_ref, degp_ref,
                  g_ref, dinv_ref):
    x = x_ref[...]
    h = _bn_rows(x, bng_ref[...], bnb_ref[...])
    h1 = _dot_t(h, w0_ref[...])
    deg = degp_ref[0, :, 0:1] + degp_ref[1, :, 0:1] + 1.0
    dinv = lax.rsqrt(deg)
    g_ref[...] = jnp.concatenate(
        [h1 * dinv[:N], jnp.zeros((N_PAD - N, D_H), jnp.float32)], axis=0)
    dinv_ref[...] = dinv


_tc_prep = pl.pallas_call(
    _tc_prep_body,
    out_shape=(jax.ShapeDtypeStruct((N_PAD, D_H), jnp.float32),
               jax.ShapeDtypeStruct((N_PAD, 1), jnp.float32)),
)


def _tc_mid_body(p_ref, gprev_ref, dinv_ref, b_ref, gam_ref, bet_ref, w_ref,
                 g_ref):
    dinv = dinv_ref[...]
    acc = p_ref[0, :N, :] + p_ref[1, :N, :] + gprev_ref[:N, :]
    out = acc * dinv[:N] + b_ref[...]
    h = jnp.maximum(_bn_rows(out, gam_ref[...], bet_ref[...]), 0.0)
    h1 = _dot_t(h, w_ref[...])
    g_ref[...] = jnp.concatenate(
        [h1 * dinv[:N], jnp.zeros((N_PAD - N, D_H), jnp.float32)], axis=0)


_tc_mid = pl.pallas_call(
    _tc_mid_body,
    out_shape=jax.ShapeDtypeStruct((N_PAD, D_H), jnp.float32),
)


def _tc_final_body(p_ref, gprev_ref, dinv_ref, b_ref, gam_ref, bet_ref,
                   batch_ref, cw1_ref, cb1_ref, cw2_ref, cb2_ref, res_ref):
    dinv = dinv_ref[...]
    acc = p_ref[0, :N, :] + p_ref[1, :N, :] + gprev_ref[:N, :]
    out = acc * dinv[:N] + b_ref[...]
    h = jnp.maximum(_bn_rows(out, gam_ref[...], bet_ref[...]), 0.0)
    seg = batch_ref[...]
    oh = (seg == lax.broadcasted_iota(jnp.int32, (1, N_GRAPHS), 1))
    oh = oh.astype(jnp.float32)
    sums = lax.dot_general(oh, h, (((0,), (0,)), ((), ())),
                           precision=lax.Precision.HIGHEST,
                           preferred_element_type=jnp.float32)
    cnt = jnp.sum(oh, axis=0)[:, None]
    pooled = sums / jnp.maximum(cnt, 1.0)
    hc = jnp.maximum(_dot_t(pooled, cw1_ref[...]) + cb1_ref[...], 0.0)
    res_ref[...] = _dot_t(hc, cw2_ref[...]) + cb2_ref[...]


_tc_final = pl.pallas_call(
    _tc_final_body,
    out_shape=jax.ShapeDtypeStruct((N_GRAPHS, N_CLASSES), jnp.float32),
)


def kernel(x, edge_index, batch, bn_in_g, bn_in_b, W0, b0, g0, be0,
           W1, b1, g1, be1, W2, b2, g2, be2, cW1, cb1, cW2, cb2):
    pad = E_PAD - E
    src = jnp.concatenate([edge_index[0], jnp.full((pad,), N, jnp.int32)])
    dst = jnp.concatenate([edge_index[1], jnp.full((pad,), N, jnp.int32)])
    src0 = src[:E0].reshape(NS, K0, CHUNK)
    dst0 = dst[:E0].reshape(NS, K0, CHUNK)
    src1 = src[E0:].reshape(NS, K1, CHUNK)
    dst1 = dst[E0:].reshape(NS, K1, CHUNK)

    zeros16 = jnp.zeros((N_PAD, 16), jnp.float32)
    ones16 = jnp.ones((CHUNK, 16), jnp.float32)
    zeros64 = jnp.zeros((N_PAD, D_H), jnp.float32)
    batch2d = batch.reshape(N, 1)

    sc_degree, sc_scatter = _sc_kernels()
    degp = sc_degree(dst0, dst1, zeros16, ones16)
    gfeat, dinv = _tc_prep(x, bn_in_g, bn_in_b, W0, degp)

    p = sc_scatter(gfeat, src0, dst0, src1, dst1, zeros64)
    gfeat = _tc_mid(p, gfeat, dinv, b0, g0, be0, W1)

    p = sc_scatter(gfeat, src0, dst0, src1, dst1, zeros64)
    gfeat = _tc_mid(p, gfeat, dinv, b1, g1, be1, W2)

    p = sc_scatter(gfeat, src0, dst0, src1, dst1, zeros64)
    return _tc_final(p, gfeat, dinv, b2, g2, be2, batch2d, cW1, cb1, cW2, cb2)
